# baseline (speedup 1.0000x reference)
import jax, jax.numpy as jnp


def kernel(fields, sides, species, moves, items, abilities,
           pokemon_attributes, move_attributes,
           species_table, move_table, item_table, ability_table):
    b = fields.shape[0]
    sp = jnp.take(species_table, species, axis=0)
    mv = jnp.take(move_table, moves, axis=0)
    it = jnp.take(item_table, items, axis=0)
    ab = jnp.take(ability_table, abilities, axis=0)
    sides_f = sides.reshape(b, -1)
    parts = [fields, sides_f]
    for j in range(2):
        for i in range(6):
            parts.append(jnp.concatenate((
                sp[:, j, i], it[:, j, i], ab[:, j, i],
                pokemon_attributes[:, j, i],
                mv[:, j, i].reshape(b, -1),
                move_attributes[:, j, i].reshape(b, -1),
            ), axis=1))
    return jnp.concatenate(parts, axis=1)


# speedup vs baseline: 4.3877x; 4.3877x over previous
import functools

import jax
import jax.numpy as jnp
from jax import lax
from jax.experimental import pallas as pl
from jax.experimental.pallas import tpu as pltpu
from jax.experimental.pallas import tpu_sc as plsc

B = 16384
NSLOT = 84
NC, NS = 2, 16
NW = NC * NS
RPW = B // NW
GCH = 128
R = 256
NCOLS = 5656

_mesh = plsc.VectorSubcoreMesh(
    core_axis_name="c", subcore_axis_name="s", num_cores=NC, num_subcores=NS)


@functools.partial(
    pl.kernel,
    out_type=jax.ShapeDtypeStruct((NSLOT * B, 128), jnp.float32),
    mesh=_mesh,
    scratch_types=[
        pltpu.VMEM((RPW,), jnp.int32),
        pltpu.VMEM((RPW, 128), jnp.float32),
    ],
)
def _gather_sc(ctab, cidx, g, uidx, gbuf):
    wid = lax.axis_index("s") * NC + lax.axis_index("c")

    def slot_body(s, carry):
        base = s * B + wid * RPW
        pltpu.sync_copy(cidx.at[pl.ds(base, RPW)], uidx)
        for q in range(RPW // GCH):
            pltpu.sync_copy(ctab.at[uidx.at[pl.ds(q * GCH, GCH)]],
                            gbuf.at[pl.ds(q * GCH, GCH)])
        pltpu.sync_copy(gbuf, g.at[pl.ds(base, RPW)])
        return carry

    lax.fori_loop(0, NSLOT, slot_body, 0)


def _assemble_tc(fields_ref, sides_ref, pa_ref, ma_ref, *gs_out):
    gs = gs_out[:NSLOT]
    out_ref = gs_out[NSLOT]
    pa = pa_ref[...]
    ma = ma_ref[...]
    parts = [fields_ref[...], sides_ref[...]]
    for p in range(12):
        parts.append(gs[p][:, :64])
        parts.append(gs[60 + p][:, :16])
        parts.append(gs[72 + p][:, :16])
        parts.append(pa[:, p * 48:(p + 1) * 48])
        for k in range(4):
            parts.append(gs[12 + 4 * p + k][:, :64])
        parts.append(ma[:, p * 64:(p + 1) * 64])
    out_ref[...] = jnp.concatenate(parts, axis=1)


def kernel(fields, sides, species, moves, items, abilities,
           pokemon_attributes, move_attributes,
           species_table, move_table, item_table, ability_table):
    b = fields.shape[0]
    ctab = jnp.concatenate([
        jnp.pad(species_table, ((0, 0), (0, 64))),
        jnp.pad(move_table, ((0, 0), (0, 64))),
        jnp.pad(item_table, ((0, 0), (0, 112))),
        jnp.pad(ability_table, ((0, 0), (0, 112))),
    ], axis=0)
    n_sp = species_table.shape[0]
    n_mv = move_table.shape[0]
    n_it = item_table.shape[0]
    cidx = jnp.concatenate([
        species.reshape(b, 12).astype(jnp.int32),
        moves.reshape(b, 48).astype(jnp.int32) + n_sp,
        items.reshape(b, 12).astype(jnp.int32) + (n_sp + n_mv),
        abilities.reshape(b, 12).astype(jnp.int32) + (n_sp + n_mv + n_it),
    ], axis=1)
    cidx_sm = cidx.T.reshape(-1)

    g = _gather_sc(ctab, cidx_sm)

    gpb = b // R
    gspecs = [
        pl.BlockSpec((R, 128), functools.partial(lambda i, s: (s * gpb + i, 0), s=s))
        for s in range(NSLOT)
    ]
    return pl.pallas_call(
        _assemble_tc,
        grid=(gpb,),
        in_specs=[
            pl.BlockSpec((R, 24), lambda i: (i, 0)),
            pl.BlockSpec((R, 64), lambda i: (i, 0)),
            pl.BlockSpec((R, 576), lambda i: (i, 0)),
            pl.BlockSpec((R, 768), lambda i: (i, 0)),
        ] + gspecs,
        out_specs=pl.BlockSpec((R, NCOLS), lambda i: (i, 0)),
        out_shape=jax.ShapeDtypeStruct((b, NCOLS), jnp.float32),
    )(fields, sides.reshape(b, 64), pokemon_attributes.reshape(b, 576),
      move_attributes.reshape(b, 768), *([g] * NSLOT))


# speedup vs baseline: 4.5330x; 1.0331x over previous
import functools

import jax
import jax.numpy as jnp
from jax import lax
from jax.experimental import pallas as pl
from jax.experimental.pallas import tpu as pltpu
from jax.experimental.pallas import tpu_sc as plsc

B = 16384
NSLOT = 84
NC, NS = 2, 16
NW = NC * NS
RPW = B // NW
GCH = 128
R = 256
NCOLS = 5656

_mesh = plsc.VectorSubcoreMesh(
    core_axis_name="c", subcore_axis_name="s", num_cores=NC, num_subcores=NS)


HCH = 256
NU = NSLOT * (RPW // HCH)


@functools.partial(
    pl.kernel,
    out_type=jax.ShapeDtypeStruct((NSLOT * B, 128), jnp.float32),
    mesh=_mesh,
    scratch_types=[
        pltpu.VMEM((2, HCH), jnp.int32),
        pltpu.VMEM((3, HCH, 128), jnp.float32),
        pltpu.SemaphoreType.DMA,
        pltpu.SemaphoreType.DMA((2,)),
        pltpu.SemaphoreType.DMA((3,)),
    ],
)
def _gather_sc(ctab, cidx, g, uidx, gbuf, sem_i, sem_g, sem_s):
    wid = lax.axis_index("s") * NC + lax.axis_index("c")

    def ubase(u):
        s = u // (RPW // HCH)
        h = lax.rem(u, RPW // HCH)
        return s * B + wid * RPW + h * HCH

    pltpu.async_copy(cidx.at[pl.ds(ubase(0), HCH)], uidx.at[0], sem_i)

    def body(u, carry):
        m3 = lax.rem(u, 3)
        p2 = lax.rem(u, 2)

        @pl.when(jnp.logical_and(u >= 3, u <= NU + 2))
        def _():
            pltpu.make_async_copy(
                gbuf.at[0], g.at[pl.ds(0, HCH)], sem_s.at[m3]).wait()

        @pl.when(u < NU)
        def _():
            pltpu.make_async_copy(
                cidx.at[pl.ds(0, HCH)], uidx.at[0], sem_i).wait()
            for q in range(HCH // GCH):
                pltpu.async_copy(
                    ctab.at[uidx.at[p2, pl.ds(q * GCH, GCH)]],
                    gbuf.at[m3, pl.ds(q * GCH, GCH)], sem_g.at[p2])

        @pl.when(jnp.logical_and(u >= 1, u <= NU))
        def _():
            pm3 = lax.rem(u + 2, 3)
            pp2 = lax.rem(u + 1, 2)
            for q in range(HCH // GCH):
                pltpu.make_async_copy(
                    ctab.at[uidx.at[0, pl.ds(0, GCH)]],
                    gbuf.at[0, pl.ds(0, GCH)], sem_g.at[pp2]).wait()
            pltpu.async_copy(gbuf.at[pm3], g.at[pl.ds(ubase(u - 1), HCH)],
                             sem_s.at[pm3])

        @pl.when(u + 1 < NU)
        def _():
            pltpu.async_copy(cidx.at[pl.ds(ubase(u + 1), HCH)],
                             uidx.at[lax.rem(u + 1, 2)], sem_i)

        return carry

    lax.fori_loop(0, NU + 3, body, 0)


def _assemble_tc(fields_ref, sides_ref, pa_ref, ma_ref, *gs_out):
    gs = gs_out[:NSLOT]
    out_ref = gs_out[NSLOT]
    pa = pa_ref[...]
    ma = ma_ref[...]
    parts = [fields_ref[...], sides_ref[...]]
    for p in range(12):
        parts.append(gs[p][:, :64])
        parts.append(gs[60 + p][:, :16])
        parts.append(gs[72 + p][:, :16])
        parts.append(pa[:, p * 48:(p + 1) * 48])
        for k in range(4):
            parts.append(gs[12 + 4 * p + k][:, :64])
        parts.append(ma[:, p * 64:(p + 1) * 64])
    out_ref[...] = jnp.concatenate(parts, axis=1)


def kernel(fields, sides, species, moves, items, abilities,
           pokemon_attributes, move_attributes,
           species_table, move_table, item_table, ability_table):
    b = fields.shape[0]
    ctab = jnp.concatenate([
        jnp.pad(species_table, ((0, 0), (0, 64))),
        jnp.pad(move_table, ((0, 0), (0, 64))),
        jnp.pad(item_table, ((0, 0), (0, 112))),
        jnp.pad(ability_table, ((0, 0), (0, 112))),
    ], axis=0)
    n_sp = species_table.shape[0]
    n_mv = move_table.shape[0]
    n_it = item_table.shape[0]
    cidx = jnp.concatenate([
        species.reshape(b, 12).astype(jnp.int32),
        moves.reshape(b, 48).astype(jnp.int32) + n_sp,
        items.reshape(b, 12).astype(jnp.int32) + (n_sp + n_mv),
        abilities.reshape(b, 12).astype(jnp.int32) + (n_sp + n_mv + n_it),
    ], axis=1)
    cidx_sm = cidx.T.reshape(-1)

    g = _gather_sc(ctab, cidx_sm)

    gpb = b // R
    gspecs = [
        pl.BlockSpec((R, 128), functools.partial(lambda i, s: (s * gpb + i, 0), s=s))
        for s in range(NSLOT)
    ]
    return pl.pallas_call(
        _assemble_tc,
        grid=(gpb,),
        in_specs=[
            pl.BlockSpec((R, 24), lambda i: (i, 0)),
            pl.BlockSpec((R, 64), lambda i: (i, 0)),
            pl.BlockSpec((R, 576), lambda i: (i, 0)),
            pl.BlockSpec((R, 768), lambda i: (i, 0)),
        ] + gspecs,
        out_specs=pl.BlockSpec((R, NCOLS), lambda i: (i, 0)),
        out_shape=jax.ShapeDtypeStruct((b, NCOLS), jnp.float32),
    )(fields, sides.reshape(b, 64), pokemon_attributes.reshape(b, 576),
      move_attributes.reshape(b, 768), *([g] * NSLOT))


# speedup vs baseline: 7.2676x; 1.6033x over previous
import functools

import jax
import jax.numpy as jnp
from jax import lax
from jax.experimental import pallas as pl
from jax.experimental.pallas import tpu as pltpu
from jax.experimental.pallas import tpu_sc as plsc

B = 16384
NSLOT = 84
NC, NS = 2, 16
NW = NC * NS
RPW = B // NW
GCH = 128
R = 256
NCOLS = 5656

_mesh = plsc.VectorSubcoreMesh(
    core_axis_name="c", subcore_axis_name="s", num_cores=NC, num_subcores=NS)


HCH = 256
NU = NSLOT * (RPW // HCH)


@functools.partial(
    pl.kernel,
    out_type=jax.ShapeDtypeStruct((NSLOT * B, 128), jnp.float32),
    mesh=_mesh,
    scratch_types=[
        pltpu.VMEM((2, HCH), jnp.int32),
        pltpu.VMEM((3, HCH, 128), jnp.float32),
        pltpu.SemaphoreType.DMA,
        pltpu.SemaphoreType.DMA((2,)),
        pltpu.SemaphoreType.DMA((3,)),
        pltpu.VMEM_SHARED((2736, 128), jnp.float32),
    ],
)
def _gather_sc(ctab, cidx, g, uidx, gbuf, sem_i, sem_g, sem_s, stab):
    wid = lax.axis_index("s") * NC + lax.axis_index("c")

    @pl.when(lax.axis_index("s") == 0)
    def _():
        pltpu.sync_copy(ctab, stab)

    plsc.subcore_barrier()

    def ubase(u):
        s = u // (RPW // HCH)
        h = lax.rem(u, RPW // HCH)
        return s * B + wid * RPW + h * HCH

    pltpu.async_copy(cidx.at[pl.ds(ubase(0), HCH)], uidx.at[0], sem_i)

    def body(u, carry):
        m3 = lax.rem(u, 3)
        p2 = lax.rem(u, 2)

        @pl.when(jnp.logical_and(u >= 3, u <= NU + 2))
        def _():
            pltpu.make_async_copy(
                gbuf.at[0], g.at[pl.ds(0, HCH)], sem_s.at[m3]).wait()

        @pl.when(u < NU)
        def _():
            pltpu.make_async_copy(
                cidx.at[pl.ds(0, HCH)], uidx.at[0], sem_i).wait()
            for q in range(HCH // GCH):
                pltpu.async_copy(
                    stab.at[uidx.at[p2, pl.ds(q * GCH, GCH)]],
                    gbuf.at[m3, pl.ds(q * GCH, GCH)], sem_g.at[p2])

        @pl.when(jnp.logical_and(u >= 1, u <= NU))
        def _():
            pm3 = lax.rem(u + 2, 3)
            pp2 = lax.rem(u + 1, 2)
            for q in range(HCH // GCH):
                pltpu.make_async_copy(
                    stab.at[uidx.at[0, pl.ds(0, GCH)]],
                    gbuf.at[0, pl.ds(0, GCH)], sem_g.at[pp2]).wait()
            pltpu.async_copy(gbuf.at[pm3], g.at[pl.ds(ubase(u - 1), HCH)],
                             sem_s.at[pm3])

        @pl.when(u + 1 < NU)
        def _():
            pltpu.async_copy(cidx.at[pl.ds(ubase(u + 1), HCH)],
                             uidx.at[lax.rem(u + 1, 2)], sem_i)

        return carry

    lax.fori_loop(0, NU + 3, body, 0)


def _assemble_tc(fields_ref, sides_ref, pa_ref, ma_ref, *gs_out):
    gs = gs_out[:NSLOT]
    out_ref = gs_out[NSLOT]
    pa = pa_ref[...]
    ma = ma_ref[...]
    parts = [fields_ref[...], sides_ref[...]]
    for p in range(12):
        parts.append(gs[p][:, :64])
        parts.append(gs[60 + p][:, :16])
        parts.append(gs[72 + p][:, :16])
        parts.append(pa[:, p * 48:(p + 1) * 48])
        for k in range(4):
            parts.append(gs[12 + 4 * p + k][:, :64])
        parts.append(ma[:, p * 64:(p + 1) * 64])
    out_ref[...] = jnp.concatenate(parts, axis=1)


def kernel(fields, sides, species, moves, items, abilities,
           pokemon_attributes, move_attributes,
           species_table, move_table, item_table, ability_table):
    b = fields.shape[0]
    ctab = jnp.concatenate([
        jnp.pad(species_table, ((0, 0), (0, 64))),
        jnp.pad(move_table, ((0, 0), (0, 64))),
        jnp.pad(item_table, ((0, 0), (0, 112))),
        jnp.pad(ability_table, ((0, 0), (0, 112))),
    ], axis=0)
    n_sp = species_table.shape[0]
    n_mv = move_table.shape[0]
    n_it = item_table.shape[0]
    cidx = jnp.concatenate([
        species.reshape(b, 12).astype(jnp.int32),
        moves.reshape(b, 48).astype(jnp.int32) + n_sp,
        items.reshape(b, 12).astype(jnp.int32) + (n_sp + n_mv),
        abilities.reshape(b, 12).astype(jnp.int32) + (n_sp + n_mv + n_it),
    ], axis=1)
    cidx_sm = cidx.T.reshape(-1)

    g = _gather_sc(ctab, cidx_sm)

    gpb = b // R
    gspecs = [
        pl.BlockSpec((R, 128), functools.partial(lambda i, s: (s * gpb + i, 0), s=s))
        for s in range(NSLOT)
    ]
    return pl.pallas_call(
        _assemble_tc,
        grid=(gpb,),
        in_specs=[
            pl.BlockSpec((R, 24), lambda i: (i, 0)),
            pl.BlockSpec((R, 64), lambda i: (i, 0)),
            pl.BlockSpec((R, 576), lambda i: (i, 0)),
            pl.BlockSpec((R, 768), lambda i: (i, 0)),
        ] + gspecs,
        out_specs=pl.BlockSpec((R, NCOLS), lambda i: (i, 0)),
        out_shape=jax.ShapeDtypeStruct((b, NCOLS), jnp.float32),
    )(fields, sides.reshape(b, 64), pokemon_attributes.reshape(b, 576),
      move_attributes.reshape(b, 768), *([g] * NSLOT))


# speedup vs baseline: 7.4214x; 1.0212x over previous
import functools

import jax
import jax.numpy as jnp
from jax import lax
from jax.experimental import pallas as pl
from jax.experimental.pallas import tpu as pltpu
from jax.experimental.pallas import tpu_sc as plsc

B = 16384
NSLOT = 84
NC, NS = 2, 16
NW = NC * NS
GCH = 128
HCH = 256
R = 256
NCOLS = 5656
NHALF = 2

_mesh = plsc.VectorSubcoreMesh(
    core_axis_name="c", subcore_axis_name="s", num_cores=NC, num_subcores=NS)


def _make_gather_sc(nb):
    rpw = nb // NW
    nu = NSLOT * (rpw // HCH)

    @functools.partial(
        pl.kernel,
        out_type=jax.ShapeDtypeStruct((NSLOT * nb, 128), jnp.float32),
        mesh=_mesh,
        scratch_types=[
            pltpu.VMEM((2, HCH), jnp.int32),
            pltpu.VMEM((3, HCH, 128), jnp.float32),
            pltpu.SemaphoreType.DMA,
            pltpu.SemaphoreType.DMA((2,)),
            pltpu.SemaphoreType.DMA((3,)),
            pltpu.VMEM_SHARED((2736, 128), jnp.float32),
        ],
    )
    def _gather_sc(ctab, cidx, g, uidx, gbuf, sem_i, sem_g, sem_s, stab):
        wid = lax.axis_index("s") * NC + lax.axis_index("c")

        @pl.when(lax.axis_index("s") == 0)
        def _():
            pltpu.sync_copy(ctab, stab)

        plsc.subcore_barrier()

        def ubase(u):
            s = u // (rpw // HCH)
            h = lax.rem(u, rpw // HCH)
            return s * nb + wid * rpw + h * HCH

        pltpu.async_copy(cidx.at[pl.ds(ubase(0), HCH)], uidx.at[0], sem_i)

        def body(u, carry):
            m3 = lax.rem(u, 3)
            p2 = lax.rem(u, 2)

            @pl.when(jnp.logical_and(u >= 3, u <= nu + 2))
            def _():
                pltpu.make_async_copy(
                    gbuf.at[0], g.at[pl.ds(0, HCH)], sem_s.at[m3]).wait()

            @pl.when(u < nu)
            def _():
                pltpu.make_async_copy(
                    cidx.at[pl.ds(0, HCH)], uidx.at[0], sem_i).wait()
                for q in range(HCH // GCH):
                    pltpu.async_copy(
                        stab.at[uidx.at[p2, pl.ds(q * GCH, GCH)]],
                        gbuf.at[m3, pl.ds(q * GCH, GCH)], sem_g.at[p2])

            @pl.when(jnp.logical_and(u >= 1, u <= nu))
            def _():
                pm3 = lax.rem(u + 2, 3)
                pp2 = lax.rem(u + 1, 2)
                for q in range(HCH // GCH):
                    pltpu.make_async_copy(
                        stab.at[uidx.at[0, pl.ds(0, GCH)]],
                        gbuf.at[0, pl.ds(0, GCH)], sem_g.at[pp2]).wait()
                pltpu.async_copy(gbuf.at[pm3], g.at[pl.ds(ubase(u - 1), HCH)],
                                 sem_s.at[pm3])

            @pl.when(u + 1 < nu)
            def _():
                pltpu.async_copy(cidx.at[pl.ds(ubase(u + 1), HCH)],
                                 uidx.at[lax.rem(u + 1, 2)], sem_i)

            return carry

        lax.fori_loop(0, nu + 3, body, 0)

    return _gather_sc


_gather_half = _make_gather_sc(B // NHALF)


def _assemble(fields_ref, sides_ref, pa_ref, ma_ref, *gs_out):
    gs = gs_out[:NSLOT]
    out_ref = gs_out[NSLOT]
    out_ref[:, 0:24] = fields_ref[...]
    out_ref[:, 24:88] = sides_ref[...]
    for p in range(12):
        c = 88 + p * 464
        out_ref[:, c:c + 64] = gs[p][:, :64]
        out_ref[:, c + 64:c + 80] = gs[60 + p][:, :16]
        out_ref[:, c + 80:c + 96] = gs[72 + p][:, :16]
        out_ref[:, c + 96:c + 144] = pa_ref[:, p * 48:(p + 1) * 48]
        for k in range(4):
            out_ref[:, c + 144 + k * 64:c + 208 + k * 64] = \
                gs[12 + 4 * p + k][:, :64]
        out_ref[:, c + 400:c + 464] = ma_ref[:, p * 64:(p + 1) * 64]


def _assemble_cont(prev_ref, *rest):
    _assemble(*rest)


def kernel(fields, sides, species, moves, items, abilities,
           pokemon_attributes, move_attributes,
           species_table, move_table, item_table, ability_table):
    b = fields.shape[0]
    ctab = jnp.concatenate([
        jnp.pad(species_table, ((0, 0), (0, 64))),
        jnp.pad(move_table, ((0, 0), (0, 64))),
        jnp.pad(item_table, ((0, 0), (0, 112))),
        jnp.pad(ability_table, ((0, 0), (0, 112))),
    ], axis=0)
    n_sp = species_table.shape[0]
    n_mv = move_table.shape[0]
    n_it = item_table.shape[0]
    cidx = jnp.concatenate([
        species.reshape(b, 12).astype(jnp.int32),
        moves.reshape(b, 48).astype(jnp.int32) + n_sp,
        items.reshape(b, 12).astype(jnp.int32) + (n_sp + n_mv),
        abilities.reshape(b, 12).astype(jnp.int32) + (n_sp + n_mv + n_it),
    ], axis=1)

    sides2 = sides.reshape(b, 64)
    pa2 = pokemon_attributes.reshape(b, 576)
    ma2 = move_attributes.reshape(b, 768)

    b2 = b // NHALF
    gpb = b2 // R
    out = None
    for h in range(NHALF):
        cidx_h = cidx[h * b2:(h + 1) * b2].T.reshape(-1)
        g_h = _gather_half(ctab, cidx_h)

        off = h * gpb
        dspec = lambda w, o=off: pl.BlockSpec((R, w), lambda i, o=o: (i + o, 0))
        gspecs = [
            pl.BlockSpec((R, 128),
                         functools.partial(lambda i, s: (s * gpb + i, 0), s=s))
            for s in range(NSLOT)
        ]
        in_specs = [dspec(24), dspec(64), dspec(576), dspec(768)] + gspecs
        args = (fields, sides2, pa2, ma2, *([g_h] * NSLOT))
        if h == 0:
            body = _assemble
            aliases = {}
        else:
            body = _assemble_cont
            in_specs = [pl.BlockSpec(memory_space=pl.ANY)] + in_specs
            args = (out, *args)
            aliases = {0: 0}
        out = pl.pallas_call(
            body,
            grid=(gpb,),
            in_specs=in_specs,
            out_specs=pl.BlockSpec((R, NCOLS), lambda i, o=off: (i + o, 0)),
            out_shape=jax.ShapeDtypeStruct((b, NCOLS), jnp.float32),
            input_output_aliases=aliases,
            compiler_params=pltpu.CompilerParams(
                vmem_limit_bytes=63 * 1024 * 1024),
        )(*args)
    return out
